# baseline (speedup 1.0000x reference)
import functools

import jax
import jax.numpy as jnp
from jax.experimental import pallas as pl
from jax.experimental.pallas import tpu as pltpu


def _se_kernel(x_ref, w1_ref, b1_ref, w2_ref, b2_ref, o_ref, *, inv_hw):
    x = x_ref[...]
    C = x.shape[0]
    pooled = (jnp.sum(x, axis=(1, 2)) * inv_hw).reshape(C, 1)
    h = jnp.dot(w1_ref[...], pooled, preferred_element_type=jnp.float32)
    h = jnp.maximum(h + b1_ref[...], 0.0)
    g = jnp.dot(w2_ref[...], h, preferred_element_type=jnp.float32)
    g = jax.nn.sigmoid(g + b2_ref[...])
    o_ref[...] = x * g[:, :, None]


def kernel(x, w1, b1, w2, b2):
    B, C, H, W = x.shape
    Ch = w1.shape[0]
    itemsize = jnp.dtype(x.dtype).itemsize

    w1f = jnp.asarray(w1, jnp.float32)
    b1c = jnp.asarray(b1, jnp.float32).reshape(Ch, 1)
    w2f = jnp.asarray(w2, jnp.float32)
    b2c = jnp.asarray(b2, jnp.float32).reshape(C, 1)

    w_pad = ((W + 127) // 128) * 128
    slab_bytes = C * H * w_pad * itemsize
    vmem_limit = int(min(100 << 20, 4 * slab_bytes + (8 << 20)))

    out = pl.pallas_call(
        functools.partial(_se_kernel, inv_hw=1.0 / (H * W)),
        out_shape=jax.ShapeDtypeStruct((B, C, H, W), x.dtype),
        grid=(B,),
        in_specs=[pl.BlockSpec((None, C, H, W), lambda b: (b, 0, 0, 0)),
                  pl.BlockSpec((Ch, C), lambda b: (0, 0)),
                  pl.BlockSpec((Ch, 1), lambda b: (0, 0)),
                  pl.BlockSpec((C, Ch), lambda b: (0, 0)),
                  pl.BlockSpec((C, 1), lambda b: (0, 0))],
        out_specs=pl.BlockSpec((None, C, H, W), lambda b: (b, 0, 0, 0)),
        compiler_params=pltpu.CompilerParams(
            dimension_semantics=("parallel",),
            vmem_limit_bytes=vmem_limit),
        cost_estimate=pl.CostEstimate(
            flops=2 * B * H * W * C + 4 * B * C * Ch,
            transcendentals=B * C,
            bytes_accessed=2 * B * C * H * w_pad * itemsize),
    )(x, w1f, b1c, w2f, b2c)
    return out


# speedup vs baseline: 6.0159x; 6.0159x over previous
import functools

import jax
import jax.numpy as jnp
from jax.experimental import pallas as pl
from jax.experimental.pallas import tpu as pltpu


def _se_kernel(x_ref, w1t_ref, b1_ref, w2t_ref, b2_ref, o_ref, *, inv_hw):
    x = x_ref[...]
    H, W, C = x.shape
    pooled = (jnp.sum(x, axis=(0, 1)) * inv_hw).reshape(1, C)
    h = jnp.dot(pooled, w1t_ref[...], preferred_element_type=jnp.float32)
    h = jnp.maximum(h + b1_ref[...], 0.0)
    g = jnp.dot(h, w2t_ref[...], preferred_element_type=jnp.float32)
    g = jax.nn.sigmoid(g + b2_ref[...])
    o_ref[...] = x * g[0][None, None, :]


def kernel(x, w1, b1, w2, b2):
    B, C, H, W = x.shape
    Ch = w1.shape[0]
    itemsize = jnp.dtype(x.dtype).itemsize

    xt = jnp.transpose(x, (0, 2, 3, 1))
    w1t = jnp.asarray(w1, jnp.float32).T
    b1r = jnp.asarray(b1, jnp.float32).reshape(1, Ch)
    w2t = jnp.asarray(w2, jnp.float32).T
    b2r = jnp.asarray(b2, jnp.float32).reshape(1, C)

    slab_bytes = H * W * C * itemsize
    vmem_limit = int(min(64 << 20, 4 * slab_bytes + (8 << 20)))

    out_t = pl.pallas_call(
        functools.partial(_se_kernel, inv_hw=1.0 / (H * W)),
        out_shape=jax.ShapeDtypeStruct((B, H, W, C), x.dtype),
        grid=(B,),
        in_specs=[pl.BlockSpec((None, H, W, C), lambda b: (b, 0, 0, 0)),
                  pl.BlockSpec((C, Ch), lambda b: (0, 0)),
                  pl.BlockSpec((1, Ch), lambda b: (0, 0)),
                  pl.BlockSpec((Ch, C), lambda b: (0, 0)),
                  pl.BlockSpec((1, C), lambda b: (0, 0))],
        out_specs=pl.BlockSpec((None, H, W, C), lambda b: (b, 0, 0, 0)),
        compiler_params=pltpu.CompilerParams(
            dimension_semantics=("parallel",),
            vmem_limit_bytes=vmem_limit),
        cost_estimate=pl.CostEstimate(
            flops=2 * B * H * W * C + 4 * B * C * Ch,
            transcendentals=B * C,
            bytes_accessed=2 * B * C * H * W * itemsize),
    )(xt, w1t, b1r, w2t, b2r)
    return jnp.transpose(out_t, (0, 3, 1, 2))
